# baseline (speedup 1.0000x reference)
import functools

import jax
import jax.numpy as jnp
from jax import lax
from jax.experimental import pallas as pl
from jax.experimental.pallas import tpu as pltpu
from jax.experimental.pallas import tpu_sc as plsc

_NUM_CORES = 2
_NUM_SUBCORES = 16
_NW = _NUM_CORES * _NUM_SUBCORES

_CHUNK = 128


def _make_sc_gather(B, d1, d2, d3):
    b_per_w = B // _NW
    n_chunk = b_per_w // _CHUNK
    mesh = plsc.VectorSubcoreMesh(
        core_axis_name="c", subcore_axis_name="s",
        num_cores=_NUM_CORES, num_subcores=_NUM_SUBCORES)

    @functools.partial(
        pl.kernel,
        mesh=mesh,
        out_type=(
            jax.ShapeDtypeStruct((B, d1), jnp.float32),
            jax.ShapeDtypeStruct((B, d2), jnp.float32),
            jax.ShapeDtypeStruct((B, d3), jnp.float32),
        ),
        scratch_types=[
            pltpu.VMEM((n_chunk, _CHUNK), jnp.int32),
            pltpu.VMEM((n_chunk, _CHUNK), jnp.int32),
            pltpu.VMEM((n_chunk, _CHUNK), jnp.int32),
            pltpu.VMEM((b_per_w, d1), jnp.float32),
            pltpu.VMEM((b_per_w, d2), jnp.float32),
            pltpu.VMEM((b_per_w, d3), jnp.float32),
            pltpu.SemaphoreType.DMA,
        ],
        compiler_params=pltpu.CompilerParams(use_tc_tiling_on_sc=False),
    )
    def gather_k(x1h, x2h, x3h, e1t, e2t, e3t, o1h, o2h, o3h,
                 i1, i2, i3, r1, r2, r3, sem):
        wid = lax.axis_index("s") * _NUM_CORES + lax.axis_index("c")
        base = wid * b_per_w
        row0 = wid * n_chunk
        pltpu.sync_copy(x1h.at[pl.ds(row0, n_chunk)], i1)
        pltpu.sync_copy(x2h.at[pl.ds(row0, n_chunk)], i2)
        pltpu.sync_copy(x3h.at[pl.ds(row0, n_chunk)], i3)
        copies = []
        for c in range(n_chunk):
            dst = pl.ds(c * _CHUNK, _CHUNK)
            copies.append(pltpu.async_copy(e1t.at[i1.at[c]], r1.at[dst], sem))
            copies.append(pltpu.async_copy(e2t.at[i2.at[c]], r2.at[dst], sem))
            copies.append(pltpu.async_copy(e3t.at[i3.at[c]], r3.at[dst], sem))
        for cp in copies:
            cp.wait()
        pltpu.sync_copy(r1, o1h.at[pl.ds(base, b_per_w)])
        pltpu.sync_copy(r2, o2h.at[pl.ds(base, b_per_w)])
        pltpu.sync_copy(r3, o3h.at[pl.ds(base, b_per_w)])

    return gather_k


def _mlp_body(e1r, e2r, e3r, w1a, w1b, w1c, b1r, w2, b2r, w3t, b3r, outr):
    f32, bf16 = jnp.float32, jnp.bfloat16

    def mm(a, b):
        return jnp.dot(a.astype(bf16), b.astype(bf16), preferred_element_type=f32)

    h = (mm(e1r[...], w1a[...]) + mm(e2r[...], w1b[...])
         + mm(e3r[...], w1c[...]))
    h = jnp.maximum(h + b1r[...], 0.0)
    h2 = jnp.maximum(mm(h, w2[...]) + b2r[...], 0.0)
    out_t = jax.lax.dot_general(
        w3t[...].astype(bf16), h2.astype(bf16),
        (((1,), (1,)), ((), ())), preferred_element_type=f32)
    outr[...] = out_t + b3r[...]


def _mlp_call(e1, e2, e3, W1a, W1b, W1c, b1, W2, b2, W3t, b3):
    B = e1.shape[0]
    n_class = W3t.shape[0]
    BB = 1024
    grid = (B // BB,)

    def batch_spec(d):
        return pl.BlockSpec((BB, d), lambda i: (i, 0))

    def full_spec(a):
        return pl.BlockSpec(a.shape, lambda i: (0,) * a.ndim)

    return pl.pallas_call(
        _mlp_body,
        grid=grid,
        in_specs=[
            batch_spec(e1.shape[1]), batch_spec(e2.shape[1]), batch_spec(e3.shape[1]),
            full_spec(W1a), full_spec(W1b), full_spec(W1c), full_spec(b1),
            full_spec(W2), full_spec(b2), full_spec(W3t), full_spec(b3),
        ],
        out_specs=pl.BlockSpec((n_class, BB), lambda i: (0, i)),
        out_shape=jax.ShapeDtypeStruct((n_class, B), jnp.float32),
    )(e1, e2, e3, W1a, W1b, W1c, b1, W2, b2, W3t, b3)


def kernel(x1, x2, x3, E1, E2, E3, W1, b1, W2, b2, W3, b3):
    B = x1.shape[0]
    d1, d2, d3 = E1.shape[1], E2.shape[1], E3.shape[1]
    gather_fn = _make_sc_gather(B, d1, d2, d3)
    x1r = x1.reshape(B // _CHUNK, _CHUNK)
    x2r = x2.reshape(B // _CHUNK, _CHUNK)
    x3r = x3.reshape(B // _CHUNK, _CHUNK)
    e1, e2, e3 = gather_fn(x1r, x2r, x3r, E1, E2, E3)
    W1a, W1b, W1c = W1[:d1], W1[d1:d1 + d2], W1[d1 + d2:]
    out_t = _mlp_call(e1, e2, e3, W1a, W1b, W1c,
                      b1.reshape(1, -1), W2, b2.reshape(1, -1), W3.T,
                      b3.reshape(-1, 1))
    return out_t.T


# speedup vs baseline: 1.6115x; 1.6115x over previous
import functools

import jax
import jax.numpy as jnp
from jax import lax
from jax.experimental import pallas as pl
from jax.experimental.pallas import tpu as pltpu
from jax.experimental.pallas import tpu_sc as plsc

_NC = 2
_NS = 16
_NW = _NC * _NS

_T = 16
_RING = 4


def _make_sc_gather(B, n1, n2, n3, d1, d2, d3):
    b_per_w = B // _NW
    n_iter = b_per_w // _T
    mesh = plsc.VectorSubcoreMesh(
        core_axis_name="c", subcore_axis_name="s",
        num_cores=_NC, num_subcores=_NS)

    @functools.partial(
        pl.kernel,
        mesh=mesh,
        out_type=(
            jax.ShapeDtypeStruct((d1, B), jnp.float32),
            jax.ShapeDtypeStruct((d2 // 2, B), jnp.int32),
            jax.ShapeDtypeStruct((d3 // 2, B), jnp.int32),
        ),
        scratch_types=[
            pltpu.VMEM((b_per_w,), jnp.int32),
            pltpu.VMEM((b_per_w,), jnp.int32),
            pltpu.VMEM((b_per_w,), jnp.int32),
            pltpu.VMEM((_RING, d1, 128), jnp.float32),
            pltpu.VMEM((_RING, d2 // 2, 128), jnp.int32),
            pltpu.VMEM((d3 // 2, n3), jnp.int32),
            pltpu.VMEM((d1, b_per_w // 2), jnp.float32),
            pltpu.VMEM((d2 // 2, b_per_w // 2), jnp.int32),
            pltpu.VMEM((d3 // 2, b_per_w // 2), jnp.int32),
            pltpu.SemaphoreType.DMA((_RING,)),
            pltpu.SemaphoreType.DMA((_RING,)),
            pltpu.SemaphoreType.DMA,
        ],
        compiler_params=pltpu.CompilerParams(needs_layout_passes=False),
    )
    def gather_k(x1h, x2h, x3h, e1th, e2ith, e3ith, o1h, o2h, o3h,
                 xv1, xv2, xv3, c1, c2, e3tab, o1, o2, o3, sem1, sem2, sem):
        wid = lax.axis_index("s") * _NC + lax.axis_index("c")
        base = wid * b_per_w
        pltpu.sync_copy(x1h.at[pl.ds(base, b_per_w)], xv1)
        pltpu.sync_copy(x2h.at[pl.ds(base, b_per_w)], xv2)
        pltpu.sync_copy(x3h.at[pl.ds(base, b_per_w)], xv3)
        pltpu.sync_copy(e3ith, e3tab)

        iota16 = lax.iota(jnp.int32, 16)

        def fire(idx1, idx2, k):
            s = k % _RING
            c1off = pl.multiple_of((idx1 >> 7) * 128, 128)
            c2off = pl.multiple_of((idx2 >> 7) * 128, 128)
            h1 = pltpu.async_copy(
                e1th.at[:, pl.ds(c1off, 128)], c1.at[s], sem1.at[s])
            h2 = pltpu.async_copy(
                e2ith.at[:, pl.ds(c2off, 128)], c2.at[s], sem2.at[s])
            return h1, h2

        def body(t, carry, off):
            v1 = xv1[pl.ds(off + t * _T, _T)]
            v2 = xv2[pl.ds(off + t * _T, _T)]
            v3 = xv3[pl.ds(off + t * _T, _T)]
            i1 = [v1[j] for j in range(_T)]
            i2 = [v2[j] for j in range(_T)]
            i3 = [v3[j] for j in range(_T)]
            handles = {}
            for k in range(_RING):
                handles[k] = fire(i1[k], i2[k], k)
            for k in range(_T):
                s = k % _RING
                h1, h2 = handles.pop(k)
                h1.wait()
                h2.wait()
                bv = jnp.broadcast_to(t * _T + k, (16,))
                l1 = jnp.broadcast_to(i1[k] & 127, (16,))
                g0 = plsc.load_gather(c1.at[s], [iota16, l1])
                g1 = plsc.load_gather(c1.at[s], [iota16 + 16, l1])
                plsc.store_scatter(o1, [iota16, bv], g0)
                plsc.store_scatter(o1, [iota16 + 16, bv], g1)
                l2 = jnp.broadcast_to(i2[k] & 127, (16,))
                r2 = plsc.load_gather(c2.at[s], [iota16, l2])
                plsc.store_scatter(o2, [iota16, bv], r2)
                l3 = jnp.broadcast_to(i3[k], (16,))
                r3a = plsc.load_gather(e3tab, [iota16, l3])
                r3b = plsc.load_gather(e3tab, [iota16 + 16, l3])
                plsc.store_scatter(o3, [iota16, bv], r3a)
                plsc.store_scatter(o3, [iota16 + 16, bv], r3b)
                if k + _RING < _T:
                    handles[k + _RING] = fire(i1[k + _RING], i2[k + _RING],
                                              k + _RING)
            return carry

        half = b_per_w // 2
        for h in range(2):
            lax.fori_loop(0, n_iter // 2,
                          functools.partial(body, off=h * half), 0)
            pltpu.sync_copy(o1, o1h.at[:, pl.ds(base + h * half, half)])
            pltpu.sync_copy(o2, o2h.at[:, pl.ds(base + h * half, half)])
            pltpu.sync_copy(o3, o3h.at[:, pl.ds(base + h * half, half)])

    return gather_k


def _mlp_body(e1tr, e2r, e3r, w1at, w1bt, w1ct, b1r, w2t, b2r, w3t, b3r, outr):
    f32, bf16 = jnp.float32, jnp.bfloat16

    def dg(a, b, dims):
        return jax.lax.dot_general(a.astype(bf16), b.astype(bf16),
                                   (dims, ((), ())),
                                   preferred_element_type=f32)

    h1t = (dg(w1at[...], e1tr[...], ((1,), (0,)))
           + dg(w1bt[...], e2r[...], ((1,), (0,)))
           + dg(w1ct[...], e3r[...], ((1,), (0,))))
    h1t = jnp.maximum(h1t + b1r[...], 0.0)
    h2t = jnp.maximum(dg(w2t[...], h1t, ((1,), (0,))) + b2r[...], 0.0)
    outr[...] = dg(w3t[...], h2t, ((1,), (0,))) + b3r[...]


def _mlp_call(e1t, e2, e3, w1at, w1bt, w1ct, b1, w2t, b2, w3t, b3):
    B = e1t.shape[1]
    n_class = w3t.shape[0]
    BB = 1024
    grid = (B // BB,)

    def full_spec(a):
        return pl.BlockSpec(a.shape, lambda i: (0,) * a.ndim)

    return pl.pallas_call(
        _mlp_body,
        grid=grid,
        in_specs=[
            pl.BlockSpec((e1t.shape[0], BB), lambda i: (0, i)),
            pl.BlockSpec((e2.shape[0], BB), lambda i: (0, i)),
            pl.BlockSpec((e3.shape[0], BB), lambda i: (0, i)),
            full_spec(w1at), full_spec(w1bt), full_spec(w1ct), full_spec(b1),
            full_spec(w2t), full_spec(b2), full_spec(w3t), full_spec(b3),
        ],
        out_specs=pl.BlockSpec((n_class, BB), lambda i: (0, i)),
        out_shape=jax.ShapeDtypeStruct((n_class, B), jnp.float32),
    )(e1t, e2, e3, w1at, w1bt, w1ct, b1, w2t, b2, w3t, b3)


def kernel(x1, x2, x3, E1, E2, E3, W1, b1, W2, b2, W3, b3):
    B = x1.shape[0]
    (n1, d1), (n2, d2), (n3, d3) = E1.shape, E2.shape, E3.shape
    E2i = lax.bitcast_convert_type(
        E2.astype(jnp.bfloat16).reshape(n2, d2 // 2, 2), jnp.int32)
    E3i = lax.bitcast_convert_type(
        E3.astype(jnp.bfloat16).reshape(n3, d3 // 2, 2), jnp.int32)
    gather_fn = _make_sc_gather(B, n1, n2, n3, d1, d2, d3)
    e1t, e2p, e3p = gather_fn(x1, x2, x3, E1.T, E2i.T, E3i.T)

    def unpack_t(ep, d):
        eb = lax.bitcast_convert_type(ep, jnp.bfloat16)
        return jnp.transpose(eb, (0, 2, 1)).reshape(d, B)

    e2 = unpack_t(e2p, d2)
    e3 = unpack_t(e3p, d3)
    w1at = W1[:d1].T
    w1bt = W1[d1:d1 + d2].T
    w1ct = W1[d1 + d2:].T
    out_t = _mlp_call(e1t, e2, e3, w1at, w1bt, w1ct,
                      b1.reshape(-1, 1), W2.T, b2.reshape(-1, 1), W3.T,
                      b3.reshape(-1, 1))
    return out_t.T
